# baseline (speedup 1.0000x reference)
import functools

import jax
import jax.numpy as jnp
from jax.experimental import pallas as pl
from jax.experimental.pallas import tpu as pltpu

K = 64


def _scan_body(states_ref, vals_ref, seq_ref, out_ref):
    T = states_ref.shape[1]
    b = pl.program_id(0)
    sl = seq_ref[b, 0]

    ones_row = jnp.ones((1, K), dtype=jnp.float32)

    def lse_row(row):
        m = jnp.max(row, axis=1, keepdims=True)
        return m + jnp.log(jnp.sum(jnp.exp(row - m), axis=1, keepdims=True))

    alpha0 = vals_ref[0, 0].reshape(1, K)
    ans0 = lse_row(alpha0)
    u0 = states_ref[0, 0]

    def step(t, carry):
        alpha, ans, u = carry
        v = states_ref[0, t]
        trans = jax.lax.dot_general(
            u, v, (((1,), (1,)), ((), ())),
            precision=jax.lax.Precision.HIGHEST,
            preferred_element_type=jnp.float32)
        alpha_col = jax.lax.dot_general(
            alpha, ones_row, (((0,), (0,)), ((), ())),
            precision=jax.lax.Precision.HIGHEST,
            preferred_element_type=jnp.float32)
        m_mat = trans + alpha_col
        m = jnp.max(m_mat, axis=0, keepdims=True)
        s = jnp.sum(jnp.exp(m_mat - m), axis=0, keepdims=True)
        e = vals_ref[0, t].reshape(1, K)
        alpha_new = e + m + jnp.log(s)
        ans_new = jnp.where(sl - 1 == t, lse_row(alpha_new), ans)
        return alpha_new, ans_new, v

    _, ans, _ = jax.lax.fori_loop(1, T, step, (alpha0, ans0, u0))
    out_ref[b, 0] = ans[0, 0]


def _run_scan(states, vals, seq_lens, interpret=False):
    B, T, _, E = states.shape
    return pl.pallas_call(
        _scan_body,
        grid=(B,),
        in_specs=[
            pl.BlockSpec((1, T, K, E), lambda b: (b, 0, 0, 0)),
            pl.BlockSpec((1, T, K), lambda b: (b, 0, 0)),
            pl.BlockSpec(memory_space=pltpu.SMEM),
        ],
        out_specs=pl.BlockSpec(memory_space=pltpu.SMEM),
        out_shape=jax.ShapeDtypeStruct((B, 1), jnp.float32),
        interpret=interpret,
    )(states, vals, seq_lens.reshape(B, 1))


@jax.jit
def kernel(state_matrix, emission_potentials, seq_lens, sum_size):
    B, T, N = emission_potentials.shape
    vals, idx = jax.lax.top_k(emission_potentials, K)
    states = jnp.take(state_matrix, idx.reshape(-1), axis=0)
    states = states.reshape(B, T, K, -1)
    out = _run_scan(states, vals, seq_lens)
    return out.reshape(B)


# speedup vs baseline: 1.2749x; 1.2749x over previous
import jax
import jax.numpy as jnp
from jax.experimental import pallas as pl
from jax.experimental.pallas import tpu as pltpu

K = 64
B = 32


def _scan_body(u_ref, v_ref, vals_ref, seq_ref, out_ref, alpha_ref, last_ref):
    t = pl.program_id(0)
    T = pl.num_programs(0)

    @pl.when(t == 0)
    def _init():
        a0 = vals_ref[0]
        alpha_ref[:, :] = a0
        last_ref[:, :] = a0

    ones_row = jnp.ones((1, K), dtype=jnp.float32)

    @pl.when(t > 0)
    def _step():
        for b in range(B):
            u = u_ref[b, 0]
            v = v_ref[b, 0]
            trans = jax.lax.dot_general(
                u, v, (((1,), (1,)), ((), ())),
                preferred_element_type=jnp.float32)
            a = alpha_ref[pl.ds(b, 1), :]
            amax = jnp.max(a, axis=1, keepdims=True)
            acol = jax.lax.dot_general(
                a - amax, ones_row, (((0,), (0,)), ((), ())),
                precision=jax.lax.Precision.HIGHEST,
                preferred_element_type=jnp.float32)
            mm = trans + acol
            m = jnp.max(mm, axis=0, keepdims=True)
            p = jnp.exp(mm - m)
            s = jax.lax.dot_general(
                ones_row, p, (((1,), (0,)), ((), ())),
                precision=jax.lax.Precision.HIGHEST,
                preferred_element_type=jnp.float32)
            e = vals_ref[0, pl.ds(b, 1), :]
            alpha_new = e + amax + m + jnp.log(s)
            alpha_ref[pl.ds(b, 1), :] = alpha_new
            sl = seq_ref[b, 0]
            last_ref[pl.ds(b, 1), :] = jnp.where(
                sl - 1 == t, alpha_new, last_ref[pl.ds(b, 1), :])

    @pl.when(t == T - 1)
    def _fin():
        la = last_ref[:, :]
        m = jnp.max(la, axis=1, keepdims=True)
        lse = m + jnp.log(jnp.sum(jnp.exp(la - m), axis=1, keepdims=True))
        for b in range(B):
            out_ref[b, 0] = lse[b, 0]


def _run_scan(states, vals, seq_lens, interpret=False):
    Bs, T, Ks, E = states.shape

    def im_u(t):
        return (0, jnp.maximum(t - 1, 0), 0, 0)

    return pl.pallas_call(
        _scan_body,
        grid=(T,),
        in_specs=[
            pl.BlockSpec((Bs, 1, Ks, E), im_u),
            pl.BlockSpec((Bs, 1, Ks, E), lambda t: (0, t, 0, 0)),
            pl.BlockSpec((1, Bs, Ks), lambda t: (t, 0, 0)),
            pl.BlockSpec(memory_space=pltpu.SMEM),
        ],
        out_specs=pl.BlockSpec(memory_space=pltpu.SMEM),
        out_shape=jax.ShapeDtypeStruct((Bs, 1), jnp.float32),
        scratch_shapes=[
            pltpu.VMEM((Bs, Ks), jnp.float32),
            pltpu.VMEM((Bs, Ks), jnp.float32),
        ],
        interpret=interpret,
    )(states, states, jnp.swapaxes(vals, 0, 1), seq_lens.reshape(Bs, 1))


@jax.jit
def kernel(state_matrix, emission_potentials, seq_lens, sum_size):
    Bs, T, N = emission_potentials.shape
    vals, idx = jax.lax.top_k(emission_potentials, K)
    states = jnp.take(state_matrix, idx.reshape(-1), axis=0)
    states = states.reshape(Bs, T, K, -1)
    out = _run_scan(states, vals, seq_lens)
    return out.reshape(Bs)


# speedup vs baseline: 1.6419x; 1.2878x over previous
import functools

import jax
import jax.numpy as jnp
from jax import lax
from jax.experimental import pallas as pl
from jax.experimental.pallas import tpu as pltpu
from jax.experimental.pallas import tpu_sc as plsc

K = 64
B = 32

_SC_INFO = plsc.get_sparse_core_info()
_NW = _SC_INFO.num_cores * _SC_INFO.num_subcores


def _sc_gather(table, idx_flat, emb):
    R = idx_flat.shape[0]
    per_w = R // _NW
    CH = 128
    n_ch = per_w // CH
    mesh = plsc.VectorSubcoreMesh(core_axis_name="c", subcore_axis_name="s")

    @functools.partial(
        pl.kernel, mesh=mesh,
        out_type=jax.ShapeDtypeStruct((R, emb), jnp.float32),
        scratch_types=[
            pltpu.VMEM((CH,), jnp.int32),
            pltpu.VMEM((CH, emb), jnp.float32),
            pltpu.SemaphoreType.DMA,
        ],
    )
    def k(table_hbm, idx_hbm, out_hbm, idx_v, rows_v, sem):
        wid = lax.axis_index("s") * _SC_INFO.num_cores + lax.axis_index("c")
        base_w = wid * per_w

        def chunk(i, _):
            base = base_w + i * CH
            pltpu.sync_copy(idx_hbm.at[pl.ds(base, CH)], idx_v)
            pltpu.async_copy(table_hbm.at[idx_v], rows_v, sem).wait()
            pltpu.sync_copy(rows_v, out_hbm.at[pl.ds(base, CH)])
            return _

        lax.fori_loop(0, n_ch, chunk, None)

    return k(table, idx_flat)


def _scan_body(u_ref, v_ref, vals_ref, seq_ref, out_ref, alpha_ref, last_ref):
    t = pl.program_id(0)
    T = pl.num_programs(0)

    @pl.when(t == 0)
    def _init():
        a0 = vals_ref[0]
        alpha_ref[:, :] = a0
        last_ref[:, :] = a0

    ones_row = jnp.ones((1, K), dtype=jnp.float32)

    @pl.when(t > 0)
    def _step():
        for b in range(B):
            u = u_ref[b, 0]
            v = v_ref[b, 0]
            trans = jax.lax.dot_general(
                u, v, (((1,), (1,)), ((), ())),
                preferred_element_type=jnp.float32)
            a = alpha_ref[pl.ds(b, 1), :]
            amax = jnp.max(a, axis=1, keepdims=True)
            acol = jax.lax.dot_general(
                a - amax, ones_row, (((0,), (0,)), ((), ())),
                precision=jax.lax.Precision.HIGHEST,
                preferred_element_type=jnp.float32)
            mm = trans + acol
            m = jnp.max(mm, axis=0, keepdims=True)
            p = jnp.exp(mm - m)
            s = jax.lax.dot_general(
                ones_row, p, (((1,), (0,)), ((), ())),
                precision=jax.lax.Precision.HIGHEST,
                preferred_element_type=jnp.float32)
            e = vals_ref[0, pl.ds(b, 1), :]
            alpha_new = e + amax + m + jnp.log(s)
            alpha_ref[pl.ds(b, 1), :] = alpha_new
            sl = seq_ref[b, 0]
            last_ref[pl.ds(b, 1), :] = jnp.where(
                sl - 1 == t, alpha_new, last_ref[pl.ds(b, 1), :])

    @pl.when(t == T - 1)
    def _fin():
        la = last_ref[:, :]
        m = jnp.max(la, axis=1, keepdims=True)
        lse = m + jnp.log(jnp.sum(jnp.exp(la - m), axis=1, keepdims=True))
        for b in range(B):
            out_ref[b, 0] = lse[b, 0]


def _run_scan(states, vals, seq_lens, interpret=False):
    Bs, T, Ks, E = states.shape

    def im_u(t):
        return (0, jnp.maximum(t - 1, 0), 0, 0)

    return pl.pallas_call(
        _scan_body,
        grid=(T,),
        in_specs=[
            pl.BlockSpec((Bs, 1, Ks, E), im_u),
            pl.BlockSpec((Bs, 1, Ks, E), lambda t: (0, t, 0, 0)),
            pl.BlockSpec((1, Bs, Ks), lambda t: (t, 0, 0)),
            pl.BlockSpec(memory_space=pltpu.SMEM),
        ],
        out_specs=pl.BlockSpec(memory_space=pltpu.SMEM),
        out_shape=jax.ShapeDtypeStruct((Bs, 1), jnp.float32),
        scratch_shapes=[
            pltpu.VMEM((Bs, Ks), jnp.float32),
            pltpu.VMEM((Bs, Ks), jnp.float32),
        ],
        interpret=interpret,
    )(states, states, jnp.swapaxes(vals, 0, 1), seq_lens.reshape(Bs, 1))


@jax.jit
def kernel(state_matrix, emission_potentials, seq_lens, sum_size):
    Bs, T, N = emission_potentials.shape
    vals, idx = jax.lax.top_k(emission_potentials, K)
    states = _sc_gather(state_matrix, idx.reshape(-1), state_matrix.shape[1])
    states = states.reshape(Bs, T, K, -1)
    out = _run_scan(states, vals, seq_lens)
    return out.reshape(Bs)


# speedup vs baseline: 1.8078x; 1.1011x over previous
import functools

import jax
import jax.numpy as jnp
from jax import lax
from jax.experimental import pallas as pl
from jax.experimental.pallas import tpu as pltpu
from jax.experimental.pallas import tpu_sc as plsc

K = 64
B = 32
N = 2048
E = 128

_SC_INFO = plsc.get_sparse_core_info()
_NW = _SC_INFO.num_cores * _SC_INFO.num_subcores


def _sc_topk_gather(emis_flat, table, T):
    R = emis_flat.shape[0] // N
    per_w = R // _NW
    NV = N // 16
    mesh = plsc.VectorSubcoreMesh(core_axis_name="c", subcore_axis_name="s")

    @functools.partial(
        pl.kernel, mesh=mesh,
        compiler_params=pltpu.CompilerParams(needs_layout_passes=False),
        out_type=[
            jax.ShapeDtypeStruct((R * K,), jnp.float32),
            jax.ShapeDtypeStruct((R * K, E), jnp.float32),
        ],
        scratch_types=[
            pltpu.VMEM((N,), jnp.float32),
            pltpu.VMEM((N,), jnp.int32),
            pltpu.VMEM((256,), jnp.int32),
            pltpu.VMEM((N + 128,), jnp.float32),
            pltpu.VMEM((N + 128,), jnp.int32),
            pltpu.VMEM((K, E), jnp.float32),
            pltpu.SemaphoreType.DMA,
        ],
    )
    def k(emis_hbm, table_hbm, vals_hbm, states_hbm,
          rowf, keys, hist, selv, seli, rows_v, sem):
        wid = lax.axis_index("s") * _SC_INFO.num_cores + lax.axis_index("c")
        iota = lax.iota(jnp.int32, 16)

        def spl(x, dt=jnp.int32):
            return lax.broadcast_in_dim(jnp.asarray(x, dt), (16,), ())

        zero16 = jnp.zeros((16,), jnp.int32)
        one16 = jnp.ones((16,), jnp.int32)

        def do_row(r, _):
            row = wid * per_w + r
            b_idx = row // T
            t_idx = row - b_idx * T
            orow = t_idx * B + b_idx
            pltpu.sync_copy(emis_hbm.at[pl.ds(row * N, N)], rowf)

            def mk(v, c):
                x = rowf[pl.ds(v * 16, 16)]
                xi = lax.bitcast_convert_type(x, jnp.int32)
                keys[pl.ds(v * 16, 16)] = xi ^ ((xi >> 31) & jnp.int32(0x7FFFFFFF))
                return c

            lax.fori_loop(0, NV, mk, 0)

            def byte_of(key, p):
                bb = (key >> (24 - 8 * p)) & 0xFF
                if p == 0:
                    bb = bb ^ 0x80
                return bb

            sel = []
            kth = jnp.int32(K)
            for p in range(4):
                def zh(v, c):
                    hist[pl.ds(v * 16, 16)] = zero16
                    return c

                lax.fori_loop(0, 16, zh, 0)

                def hst(v, c, p=p):
                    key = keys[pl.ds(v * 16, 16)]
                    if p == 0:
                        plsc.addupdate_scatter(hist, [byte_of(key, 0)], one16)
                    else:
                        m = byte_of(key, 0) == sel[0]
                        for q in range(1, p):
                            m = jnp.logical_and(m, byte_of(key, q) == sel[q])
                        plsc.addupdate_scatter(hist, [byte_of(key, p)], one16,
                                               mask=m)
                    return c

                lax.fori_loop(0, NV, hst, 0)

                ksp = spl(kth)

                def cross(vv, carry):
                    cnt, bstar, cabove, found = carry
                    v = 15 - vv
                    rv = jnp.flip(hist[pl.ds(v * 16, 16)])
                    c = plsc.cumsum(rv) + spl(cnt)
                    m = c >= ksp
                    lane = jnp.min(jnp.where(m, iota, spl(99)))
                    has = lane < 99
                    c_at = jnp.min(jnp.where(m, c, spl(1 << 30)))
                    rv_at = jnp.sum(jnp.where(spl(lane) == iota, rv, spl(0)))
                    take = jnp.logical_and(has, jnp.logical_not(found))
                    bin_cand = v * 16 + 15 - lane
                    bstar = jnp.where(take, bin_cand, bstar)
                    cabove = jnp.where(take, c_at - rv_at, cabove)
                    found = jnp.logical_or(found, has)
                    cnt = cnt + jnp.sum(rv)
                    return cnt, bstar, cabove, found

                _cnt_f, bstar, cabove, _fnd = lax.fori_loop(
                    0, 16, cross,
                    (jnp.int32(0), jnp.int32(0), jnp.int32(0),
                     jnp.zeros((), jnp.bool_)))
                sel.append(spl(bstar))
                kth = kth - cabove

            th = (((sel[0] ^ 0x80) << 24) | (sel[1] << 16)
                  | (sel[2] << 8) | sel[3])
            th_f = lax.bitcast_convert_type(th ^ ((th >> 31) & jnp.int32(0x7FFFFFFF)),
                                            jnp.float32)

            def gtp(v, off):
                key = keys[pl.ds(v * 16, 16)]
                m = key > th
                plsc.store_compressed(seli.at[pl.ds(off, 16)], iota + spl(v * 16), mask=m)
                plsc.store_compressed(selv.at[pl.ds(off, 16)],
                                      rowf[pl.ds(v * 16, 16)], mask=m)
                return off + jnp.sum(m.astype(jnp.int32))

            r_cnt = lax.fori_loop(0, NV, gtp, jnp.int32(0))

            def eqp(v, off):
                key = keys[pl.ds(v * 16, 16)]
                m = key == th
                plsc.store_compressed(seli.at[pl.ds(off, 16)], iota + spl(v * 16), mask=m)
                plsc.store_compressed(selv.at[pl.ds(off, 16)], th_f, mask=m)
                return off + jnp.sum(m.astype(jnp.int32))

            lax.fori_loop(0, NV, eqp, r_cnt)

            pltpu.async_copy(table_hbm.at[seli.at[pl.ds(0, K)]], rows_v, sem).wait()
            pltpu.sync_copy(rows_v, states_hbm.at[pl.ds(orow * K, K)])
            pltpu.sync_copy(selv.at[pl.ds(0, K)], vals_hbm.at[pl.ds(orow * K, K)])
            return 0

        lax.fori_loop(0, per_w, do_row, 0)

    return k(emis_flat, table)


def _scan_body(u_ref, v_ref, vals_ref, seq_ref, out_ref, alpha_ref, last_ref):
    t = pl.program_id(0)
    T = pl.num_programs(0)

    @pl.when(t == 0)
    def _init():
        a0 = vals_ref[0]
        alpha_ref[:, :] = a0
        last_ref[:, :] = a0

    ones_row = jnp.ones((1, K), dtype=jnp.float32)

    @pl.when(t > 0)
    def _step():
        for b in range(B):
            u = u_ref[0, b]
            v = v_ref[0, b]
            trans = jax.lax.dot_general(
                u, v, (((1,), (1,)), ((), ())),
                preferred_element_type=jnp.float32)
            a = alpha_ref[pl.ds(b, 1), :]
            amax = jnp.max(a, axis=1, keepdims=True)
            acol = jax.lax.dot_general(
                a - amax, ones_row, (((0,), (0,)), ((), ())),
                precision=jax.lax.Precision.HIGHEST,
                preferred_element_type=jnp.float32)
            mm = trans + acol
            m = jnp.max(mm, axis=0, keepdims=True)
            p = jnp.exp(mm - m)
            s = jax.lax.dot_general(
                ones_row, p, (((1,), (0,)), ((), ())),
                precision=jax.lax.Precision.HIGHEST,
                preferred_element_type=jnp.float32)
            e = vals_ref[0, pl.ds(b, 1), :]
            alpha_new = e + amax + m + jnp.log(s)
            alpha_ref[pl.ds(b, 1), :] = alpha_new
            sl = seq_ref[b, 0]
            last_ref[pl.ds(b, 1), :] = jnp.where(
                sl - 1 == t, alpha_new, last_ref[pl.ds(b, 1), :])

    @pl.when(t == T - 1)
    def _fin():
        la = last_ref[:, :]
        m = jnp.max(la, axis=1, keepdims=True)
        lse = m + jnp.log(jnp.sum(jnp.exp(la - m), axis=1, keepdims=True))
        for b in range(B):
            out_ref[b, 0] = lse[b, 0]


def _run_scan(states_tb, vals_tb, seq_lens, interpret=False):
    T, Bs, Ks, Es = states_tb.shape

    def im_u(t):
        return (jnp.maximum(t - 1, 0), 0, 0, 0)

    return pl.pallas_call(
        _scan_body,
        grid=(T,),
        in_specs=[
            pl.BlockSpec((1, Bs, Ks, Es), im_u),
            pl.BlockSpec((1, Bs, Ks, Es), lambda t: (t, 0, 0, 0)),
            pl.BlockSpec((1, Bs, Ks), lambda t: (t, 0, 0)),
            pl.BlockSpec(memory_space=pltpu.SMEM),
        ],
        out_specs=pl.BlockSpec(memory_space=pltpu.SMEM),
        out_shape=jax.ShapeDtypeStruct((Bs, 1), jnp.float32),
        scratch_shapes=[
            pltpu.VMEM((Bs, Ks), jnp.float32),
            pltpu.VMEM((Bs, Ks), jnp.float32),
        ],
        interpret=interpret,
    )(states_tb, states_tb, vals_tb, seq_lens.reshape(Bs, 1))


@jax.jit
def kernel(state_matrix, emission_potentials, seq_lens, sum_size):
    Bs, T, Nn = emission_potentials.shape
    vals_flat, states_flat = _sc_topk_gather(
        emission_potentials.reshape(-1), state_matrix, T)
    vals_tb = vals_flat.reshape(T, Bs, K)
    states_tb = states_flat.reshape(T, Bs, K, E)
    out = _run_scan(states_tb, vals_tb, seq_lens)
    return out.reshape(Bs)


# speedup vs baseline: 2.2632x; 1.2519x over previous
import functools

import jax
import jax.numpy as jnp
from jax import lax
from jax.experimental import pallas as pl
from jax.experimental.pallas import tpu as pltpu
from jax.experimental.pallas import tpu_sc as plsc

K = 64
B = 32
N = 2048
E = 128

_SC_INFO = plsc.get_sparse_core_info()
_NW = _SC_INFO.num_cores * _SC_INFO.num_subcores


def _sc_topk_gather(emis_flat, table, T):
    R = emis_flat.shape[0] // N
    per_w = R // _NW
    NV = N // 16
    mesh = plsc.VectorSubcoreMesh(core_axis_name="c", subcore_axis_name="s")

    @functools.partial(
        pl.kernel, mesh=mesh,
        compiler_params=pltpu.CompilerParams(needs_layout_passes=False),
        out_type=[
            jax.ShapeDtypeStruct((R * K,), jnp.float32),
            jax.ShapeDtypeStruct((R * K, E), jnp.float32),
        ],
        scratch_types=[
            pltpu.VMEM((N,), jnp.float32),
            pltpu.VMEM((N,), jnp.int32),
            pltpu.VMEM((256,), jnp.int32),
            pltpu.VMEM((N + 128,), jnp.float32),
            pltpu.VMEM((N + 128,), jnp.int32),
            pltpu.VMEM((K, E), jnp.float32),
            pltpu.SemaphoreType.DMA,
        ],
    )
    def k(emis_hbm, table_hbm, vals_hbm, states_hbm,
          rowf, keys, hist, selv, seli, rows_v, sem):
        wid = lax.axis_index("s") * _SC_INFO.num_cores + lax.axis_index("c")
        iota = lax.iota(jnp.int32, 16)

        def spl(x, dt=jnp.int32):
            return lax.broadcast_in_dim(jnp.asarray(x, dt), (16,), ())

        zero16 = jnp.zeros((16,), jnp.int32)
        one16 = jnp.ones((16,), jnp.int32)

        def do_row(r, _):
            row = wid * per_w + r
            b_idx = row // T
            t_idx = row - b_idx * T
            orow = t_idx * B + b_idx
            pltpu.sync_copy(emis_hbm.at[pl.ds(row * N, N)], rowf)

            def mk(v, c):
                x = rowf[pl.ds(v * 16, 16)]
                xi = lax.bitcast_convert_type(x, jnp.int32)
                keys[pl.ds(v * 16, 16)] = xi ^ ((xi >> 31) & jnp.int32(0x7FFFFFFF))
                return c

            lax.fori_loop(0, NV, mk, 0)

            def byte_of(key, p):
                bb = (key >> (24 - 8 * p)) & 0xFF
                if p == 0:
                    bb = bb ^ 0x80
                return bb

            sel = []
            kth = jnp.int32(K)
            for p in range(4):
                def zh(v, c):
                    hist[pl.ds(v * 16, 16)] = zero16
                    return c

                lax.fori_loop(0, 16, zh, 0)

                def hst(v, c, p=p):
                    key = keys[pl.ds(v * 16, 16)]
                    if p == 0:
                        plsc.addupdate_scatter(hist, [byte_of(key, 0)], one16)
                    else:
                        m = byte_of(key, 0) == sel[0]
                        for q in range(1, p):
                            m = jnp.logical_and(m, byte_of(key, q) == sel[q])
                        plsc.addupdate_scatter(hist, [byte_of(key, p)], one16,
                                               mask=m)
                    return c

                lax.fori_loop(0, NV, hst, 0)

                ksp = spl(kth)

                def cross(vv, carry):
                    cnt, bstar, cabove, found = carry
                    v = 15 - vv
                    rv = jnp.flip(hist[pl.ds(v * 16, 16)])
                    c = plsc.cumsum(rv) + spl(cnt)
                    m = c >= ksp
                    lane = jnp.min(jnp.where(m, iota, spl(99)))
                    has = lane < 99
                    c_at = jnp.min(jnp.where(m, c, spl(1 << 30)))
                    rv_at = jnp.sum(jnp.where(spl(lane) == iota, rv, spl(0)))
                    take = jnp.logical_and(has, jnp.logical_not(found))
                    bin_cand = v * 16 + 15 - lane
                    bstar = jnp.where(take, bin_cand, bstar)
                    cabove = jnp.where(take, c_at - rv_at, cabove)
                    found = jnp.logical_or(found, has)
                    cnt = cnt + jnp.sum(rv)
                    return cnt, bstar, cabove, found

                _cnt_f, bstar, cabove, _fnd = lax.fori_loop(
                    0, 16, cross,
                    (jnp.int32(0), jnp.int32(0), jnp.int32(0),
                     jnp.zeros((), jnp.bool_)))
                sel.append(spl(bstar))
                kth = kth - cabove

            th = (((sel[0] ^ 0x80) << 24) | (sel[1] << 16)
                  | (sel[2] << 8) | sel[3])
            th_f = lax.bitcast_convert_type(th ^ ((th >> 31) & jnp.int32(0x7FFFFFFF)),
                                            jnp.float32)

            def gtp(v, off):
                key = keys[pl.ds(v * 16, 16)]
                m = key > th
                plsc.store_compressed(seli.at[pl.ds(off, 16)], iota + spl(v * 16), mask=m)
                plsc.store_compressed(selv.at[pl.ds(off, 16)],
                                      rowf[pl.ds(v * 16, 16)], mask=m)
                return off + jnp.sum(m.astype(jnp.int32))

            r_cnt = lax.fori_loop(0, NV, gtp, jnp.int32(0))

            def eqp(v, off):
                key = keys[pl.ds(v * 16, 16)]
                m = key == th
                plsc.store_compressed(seli.at[pl.ds(off, 16)], iota + spl(v * 16), mask=m)
                plsc.store_compressed(selv.at[pl.ds(off, 16)], th_f, mask=m)
                return off + jnp.sum(m.astype(jnp.int32))

            lax.fori_loop(0, NV, eqp, r_cnt)

            pltpu.async_copy(table_hbm.at[seli.at[pl.ds(0, K)]], rows_v, sem).wait()
            pltpu.sync_copy(rows_v, states_hbm.at[pl.ds(orow * K, K)])
            pltpu.sync_copy(selv.at[pl.ds(0, K)], vals_hbm.at[pl.ds(orow * K, K)])
            return 0

        lax.fori_loop(0, per_w, do_row, 0)

    return k(emis_flat, table)


def _scan_body(u_ref, v_ref, vals_ref, seq_ref, pblk_ref,
               out_ref, alpha_ref, last_ref, mm_ref):
    t = pl.program_id(0)
    T = pl.num_programs(0)

    @pl.when(t == 0)
    def _init():
        a0 = vals_ref[0]
        alpha_ref[:, :] = a0
        last_ref[:, :] = a0

    @pl.when(t > 0)
    def _step():
        for b in range(B):
            mm_ref[pl.ds(b * K, K), :] = jax.lax.dot_general(
                u_ref[0, b], v_ref[0, b], (((1,), (1,)), ((), ())),
                preferred_element_type=jnp.float32)
        pblk = pblk_ref[:, :]
        alpha = alpha_ref[:, :]
        amax = jnp.max(alpha, axis=1, keepdims=True)
        arel = alpha - amax
        acol = jax.lax.dot_general(
            pblk, arel, (((1,), (0,)), ((), ())),
            precision=jax.lax.Precision.HIGHEST,
            preferred_element_type=jnp.float32)
        mm = mm_ref[:, :] + acol
        m3 = jnp.max(mm.reshape(B, K, K), axis=1)
        mcol = jax.lax.dot_general(
            pblk, m3, (((1,), (0,)), ((), ())),
            precision=jax.lax.Precision.HIGHEST,
            preferred_element_type=jnp.float32)
        p = jnp.exp(mm - mcol)
        s = jax.lax.dot_general(
            pblk, p, (((0,), (0,)), ((), ())),
            precision=jax.lax.Precision.HIGHEST,
            preferred_element_type=jnp.float32)
        alpha_new = vals_ref[0] + amax + m3 + jnp.log(s)
        alpha_ref[:, :] = alpha_new
        mask = seq_ref[:, :] - 1 == t
        last_ref[:, :] = jnp.where(mask, alpha_new, last_ref[:, :])

    @pl.when(t == T - 1)
    def _fin():
        la = last_ref[:, :]
        m = jnp.max(la, axis=1, keepdims=True)
        lse = m + jnp.log(jnp.sum(jnp.exp(la - m), axis=1, keepdims=True))
        for b in range(B):
            out_ref[b, 0] = lse[b, 0]


def _run_scan(states_tb, vals_tb, seq_lens, interpret=False):
    T, Bs, Ks, Es = states_tb.shape
    pblk = (jax.lax.broadcasted_iota(jnp.int32, (Bs * Ks, Bs), 0) // Ks
            == jax.lax.broadcasted_iota(jnp.int32, (Bs * Ks, Bs), 1)
            ).astype(jnp.float32)

    def im_u(t):
        return (jnp.maximum(t - 1, 0), 0, 0, 0)

    return pl.pallas_call(
        _scan_body,
        grid=(T,),
        in_specs=[
            pl.BlockSpec((1, Bs, Ks, Es), im_u),
            pl.BlockSpec((1, Bs, Ks, Es), lambda t: (t, 0, 0, 0)),
            pl.BlockSpec((1, Bs, Ks), lambda t: (t, 0, 0)),
            pl.BlockSpec((Bs, 1), lambda t: (0, 0)),
            pl.BlockSpec((Bs * Ks, Bs), lambda t: (0, 0)),
        ],
        out_specs=pl.BlockSpec(memory_space=pltpu.SMEM),
        out_shape=jax.ShapeDtypeStruct((Bs, 1), jnp.float32),
        scratch_shapes=[
            pltpu.VMEM((Bs, Ks), jnp.float32),
            pltpu.VMEM((Bs, Ks), jnp.float32),
            pltpu.VMEM((Bs * Ks, Ks), jnp.float32),
        ],
        interpret=interpret,
    )(states_tb, states_tb, vals_tb, seq_lens.reshape(Bs, 1), pblk)


@jax.jit
def kernel(state_matrix, emission_potentials, seq_lens, sum_size):
    Bs, T, Nn = emission_potentials.shape
    vals_flat, states_flat = _sc_topk_gather(
        emission_potentials.reshape(-1), state_matrix, T)
    vals_tb = vals_flat.reshape(T, Bs, K)
    states_tb = states_flat.reshape(T, Bs, K, E)
    out = _run_scan(states_tb, vals_tb, seq_lens)
    return out.reshape(Bs)


# speedup vs baseline: 2.5174x; 1.1123x over previous
import functools

import jax
import jax.numpy as jnp
from jax import lax
from jax.experimental import pallas as pl
from jax.experimental.pallas import tpu as pltpu
from jax.experimental.pallas import tpu_sc as plsc

K = 64
B = 32
N = 2048
E = 128

_NC = 2
_NS = 16
_NW = _NC * _NS


def _sc_topk_gather(emis_flat, table, T):
    R = emis_flat.shape[0] // N
    per_w = R // _NW
    NV = N // 16
    mesh = plsc.VectorSubcoreMesh(core_axis_name="c", subcore_axis_name="s")

    @functools.partial(
        pl.kernel, mesh=mesh,
        compiler_params=pltpu.CompilerParams(needs_layout_passes=False),
        out_type=[
            jax.ShapeDtypeStruct((R * K,), jnp.float32),
            jax.ShapeDtypeStruct((R * K, E), jnp.float32),
        ],
        scratch_types=[
            pltpu.VMEM((N,), jnp.float32),
            pltpu.VMEM((N,), jnp.int32),
            pltpu.VMEM((256,), jnp.int32),
            pltpu.VMEM((N + 128,), jnp.float32),
            pltpu.VMEM((N + 128,), jnp.int32),
            pltpu.VMEM((K, E), jnp.float32),
            pltpu.SemaphoreType.DMA,
        ],
    )
    def k(emis_hbm, table_hbm, vals_hbm, states_hbm,
          rowf, keys, hist, selv, seli, rows_v, sem):
        wid = lax.axis_index("s") * _NC + lax.axis_index("c")
        iota = lax.iota(jnp.int32, 16)

        def spl(x, dt=jnp.int32):
            return lax.broadcast_in_dim(jnp.asarray(x, dt), (16,), ())

        zero16 = jnp.zeros((16,), jnp.int32)
        one16 = jnp.ones((16,), jnp.int32)

        def do_row(r, _):
            row = wid * per_w + r
            b_idx = row // T
            t_idx = row - b_idx * T
            orow = t_idx * B + b_idx
            pltpu.sync_copy(emis_hbm.at[pl.ds(row * N, N)], rowf)

            def mk(v, c):
                x = rowf[pl.ds(v * 16, 16)]
                xi = lax.bitcast_convert_type(x, jnp.int32)
                keys[pl.ds(v * 16, 16)] = xi ^ ((xi >> 31) & jnp.int32(0x7FFFFFFF))
                return c

            lax.fori_loop(0, NV, mk, 0)

            def byte_of(key, p):
                bb = (key >> (24 - 8 * p)) & 0xFF
                if p == 0:
                    bb = bb ^ 0x80
                return bb

            sel = []
            kth = jnp.int32(K)
            for p in range(4):
                def zh(v, c):
                    hist[pl.ds(v * 16, 16)] = zero16
                    return c

                lax.fori_loop(0, 16, zh, 0)

                def hst(v, c, p=p):
                    key = keys[pl.ds(v * 16, 16)]
                    if p == 0:
                        plsc.addupdate_scatter(hist, [byte_of(key, 0)], one16)
                    else:
                        m = byte_of(key, 0) == sel[0]
                        for q in range(1, p):
                            m = jnp.logical_and(m, byte_of(key, q) == sel[q])
                        plsc.addupdate_scatter(hist, [byte_of(key, p)], one16,
                                               mask=m)
                    return c

                lax.fori_loop(0, NV, hst, 0)

                ksp = spl(kth)

                def cross(vv, carry):
                    cnt, bstar, cabove, found = carry
                    v = 15 - vv
                    rv = jnp.flip(hist[pl.ds(v * 16, 16)])
                    c = plsc.cumsum(rv) + spl(cnt)
                    m = c >= ksp
                    lane = jnp.min(jnp.where(m, iota, spl(99)))
                    has = lane < 99
                    c_at = jnp.min(jnp.where(m, c, spl(1 << 30)))
                    rv_at = jnp.sum(jnp.where(spl(lane) == iota, rv, spl(0)))
                    take = jnp.logical_and(has, jnp.logical_not(found))
                    bin_cand = v * 16 + 15 - lane
                    bstar = jnp.where(take, bin_cand, bstar)
                    cabove = jnp.where(take, c_at - rv_at, cabove)
                    found = jnp.logical_or(found, has)
                    cnt = cnt + jnp.sum(rv)
                    return cnt, bstar, cabove, found

                _cnt_f, bstar, cabove, _fnd = lax.fori_loop(
                    0, 16, cross,
                    (jnp.int32(0), jnp.int32(0), jnp.int32(0),
                     jnp.zeros((), jnp.bool_)))
                sel.append(spl(bstar))
                kth = kth - cabove

            th = (((sel[0] ^ 0x80) << 24) | (sel[1] << 16)
                  | (sel[2] << 8) | sel[3])
            th_f = lax.bitcast_convert_type(th ^ ((th >> 31) & jnp.int32(0x7FFFFFFF)),
                                            jnp.float32)

            def gtp(v, off):
                key = keys[pl.ds(v * 16, 16)]
                m = key > th
                plsc.store_compressed(seli.at[pl.ds(off, 16)], iota + spl(v * 16), mask=m)
                plsc.store_compressed(selv.at[pl.ds(off, 16)],
                                      rowf[pl.ds(v * 16, 16)], mask=m)
                return off + jnp.sum(m.astype(jnp.int32))

            r_cnt = lax.fori_loop(0, NV, gtp, jnp.int32(0))

            def eqp(v, off):
                key = keys[pl.ds(v * 16, 16)]
                m = key == th
                plsc.store_compressed(seli.at[pl.ds(off, 16)], iota + spl(v * 16), mask=m)
                plsc.store_compressed(selv.at[pl.ds(off, 16)], th_f, mask=m)
                return off + jnp.sum(m.astype(jnp.int32))

            lax.fori_loop(0, NV, eqp, r_cnt)

            pltpu.async_copy(table_hbm.at[seli.at[pl.ds(0, K)]], rows_v, sem).wait()
            pltpu.sync_copy(rows_v, states_hbm.at[pl.ds(orow * K, K)])
            pltpu.sync_copy(selv.at[pl.ds(0, K)], vals_hbm.at[pl.ds(orow * K, K)])
            return 0

        lax.fori_loop(0, per_w, do_row, 0)

    return k(emis_flat, table)


def _scan_body(u_ref, v_ref, vals_ref, seq_ref, pblk_ref,
               out_ref, alpha_ref, last_ref, mm_ref):
    t = pl.program_id(0)
    T = pl.num_programs(0)

    @pl.when(t == 0)
    def _init():
        a0 = vals_ref[0]
        alpha_ref[:, :] = a0
        last_ref[:, :] = a0

    @pl.when(t > 0)
    def _step():
        for b in range(B):
            mm_ref[pl.ds(b * K, K), :] = jax.lax.dot_general(
                u_ref[0, b], v_ref[0, b], (((1,), (1,)), ((), ())),
                preferred_element_type=jnp.float32)
        pblk = pblk_ref[:, :]
        alpha = alpha_ref[:, :]
        amax = jnp.max(alpha, axis=1, keepdims=True)
        arel = alpha - amax
        arel_t = arel.T
        tiled = jnp.broadcast_to(
            arel_t.reshape(1, K, B), (B, K, B)).reshape(B * K, B)
        acol = jnp.sum(tiled * pblk, axis=1, keepdims=True)
        mm = mm_ref[:, :] + acol
        m3 = jnp.max(mm.reshape(B, K, K), axis=1)
        mcol = jax.lax.dot_general(
            pblk, m3, (((1,), (0,)), ((), ())),
            precision=jax.lax.Precision.HIGHEST,
            preferred_element_type=jnp.float32)
        p = jnp.exp(mm - mcol)
        s = jax.lax.dot_general(
            pblk, p, (((0,), (0,)), ((), ())),
            precision=jax.lax.Precision.HIGHEST,
            preferred_element_type=jnp.float32)
        alpha_new = vals_ref[0] + amax + m3 + jnp.log(s)
        alpha_ref[:, :] = alpha_new
        mask = seq_ref[:, :] - 1 == t
        last_ref[:, :] = jnp.where(mask, alpha_new, last_ref[:, :])

    @pl.when(t == T - 1)
    def _fin():
        la = last_ref[:, :]
        m = jnp.max(la, axis=1, keepdims=True)
        lse = m + jnp.log(jnp.sum(jnp.exp(la - m), axis=1, keepdims=True))
        for b in range(B):
            out_ref[b, 0] = lse[b, 0]


def _run_scan(states_tb, vals_tb, seq_lens, interpret=False):
    T, Bs, Ks, Es = states_tb.shape
    pblk = (jax.lax.broadcasted_iota(jnp.int32, (Bs * Ks, Bs), 0) // Ks
            == jax.lax.broadcasted_iota(jnp.int32, (Bs * Ks, Bs), 1)
            ).astype(jnp.float32)

    def im_u(t):
        return (jnp.maximum(t - 1, 0), 0, 0, 0)

    return pl.pallas_call(
        _scan_body,
        grid=(T,),
        in_specs=[
            pl.BlockSpec((1, Bs, Ks, Es), im_u),
            pl.BlockSpec((1, Bs, Ks, Es), lambda t: (t, 0, 0, 0)),
            pl.BlockSpec((1, Bs, Ks), lambda t: (t, 0, 0)),
            pl.BlockSpec((Bs, 1), lambda t: (0, 0)),
            pl.BlockSpec((Bs * Ks, Bs), lambda t: (0, 0)),
        ],
        out_specs=pl.BlockSpec(memory_space=pltpu.SMEM),
        out_shape=jax.ShapeDtypeStruct((Bs, 1), jnp.float32),
        scratch_shapes=[
            pltpu.VMEM((Bs, Ks), jnp.float32),
            pltpu.VMEM((Bs, Ks), jnp.float32),
            pltpu.VMEM((Bs * Ks, Ks), jnp.float32),
        ],
        interpret=interpret,
    )(states_tb, states_tb, vals_tb, seq_lens.reshape(Bs, 1), pblk)


@jax.jit
def kernel(state_matrix, emission_potentials, seq_lens, sum_size):
    Bs, T, Nn = emission_potentials.shape
    vals_flat, states_flat = _sc_topk_gather(
        emission_potentials.reshape(-1), state_matrix, T)
    vals_tb = vals_flat.reshape(T, Bs, K)
    states_tb = states_flat.reshape(T, Bs, K, E)
    out = _run_scan(states_tb, vals_tb, seq_lens)
    return out.reshape(Bs)


# speedup vs baseline: 2.6341x; 1.0464x over previous
import functools

import jax
import jax.numpy as jnp
from jax import lax
from jax.experimental import pallas as pl
from jax.experimental.pallas import tpu as pltpu
from jax.experimental.pallas import tpu_sc as plsc

K = 64
B = 32
N = 2048
E = 128

_NC = 2
_NS = 16
_NW = _NC * _NS


def _sc_topk_gather(emis_flat, table, T):
    R = emis_flat.shape[0] // N
    per_w = R // _NW
    NV = N // 16
    mesh = plsc.VectorSubcoreMesh(core_axis_name="c", subcore_axis_name="s")

    @functools.partial(
        pl.kernel, mesh=mesh,
        compiler_params=pltpu.CompilerParams(needs_layout_passes=False),
        out_type=[
            jax.ShapeDtypeStruct((R * K,), jnp.float32),
            jax.ShapeDtypeStruct((R * K, E), jnp.float32),
        ],
        scratch_types=[
            pltpu.VMEM((2, N), jnp.float32),
            pltpu.VMEM((N,), jnp.int32),
            pltpu.VMEM((256,), jnp.int32),
            pltpu.VMEM((2, N + 128), jnp.float32),
            pltpu.VMEM((2, N + 128), jnp.int32),
            pltpu.VMEM((2, K, E), jnp.float32),
            pltpu.SemaphoreType.DMA,
            pltpu.SemaphoreType.DMA,
            pltpu.SemaphoreType.DMA,
        ],
    )
    def k(emis_hbm, table_hbm, vals_hbm, states_hbm,
          rowf, keys, hist, selv, seli, rows_v, sem_in, sem_g, sem_out):
        wid = lax.axis_index("s") * _NC + lax.axis_index("c")
        iota = lax.iota(jnp.int32, 16)

        def spl(x, dt=jnp.int32):
            return lax.broadcast_in_dim(jnp.asarray(x, dt), (16,), ())

        zero16 = jnp.zeros((16,), jnp.int32)
        one16 = jnp.ones((16,), jnp.int32)
        row0 = wid * per_w
        last_row = row0 + per_w - 1

        pltpu.async_copy(emis_hbm.at[pl.ds(row0 * N, N)], rowf.at[0], sem_in)

        def do_row(r, ph):
            row = row0 + r
            b_idx = row // T
            t_idx = row - b_idx * T
            orow = t_idx * B + b_idx

            @pl.when(r >= 2)
            def _drain_out():
                pltpu.make_async_copy(
                    rows_v.at[ph], states_hbm.at[pl.ds(0, K)], sem_out).wait()
                pltpu.make_async_copy(
                    selv.at[ph, pl.ds(0, K)], vals_hbm.at[pl.ds(0, K)],
                    sem_out).wait()

            pltpu.make_async_copy(
                emis_hbm.at[pl.ds(0, N)], rowf.at[ph], sem_in).wait()

            @pl.when(row < last_row)
            def _prefetch():
                pltpu.async_copy(
                    emis_hbm.at[pl.ds((row + 1) * N, N)], rowf.at[1 - ph],
                    sem_in)

            def mk(v, c):
                x = rowf[ph, pl.ds(v * 16, 16)]
                xi = lax.bitcast_convert_type(x, jnp.int32)
                keys[pl.ds(v * 16, 16)] = xi ^ ((xi >> 31) & jnp.int32(0x7FFFFFFF))
                return c

            lax.fori_loop(0, NV, mk, 0)

            def byte_of(key, p):
                bb = (key >> (24 - 8 * p)) & 0xFF
                if p == 0:
                    bb = bb ^ 0x80
                return bb

            sel = []
            kth = jnp.int32(K)
            for p in range(4):
                def zh(v, c):
                    hist[pl.ds(v * 16, 16)] = zero16
                    return c

                lax.fori_loop(0, 16, zh, 0)

                def hst(v, c, p=p):
                    key = keys[pl.ds(v * 16, 16)]
                    if p == 0:
                        plsc.addupdate_scatter(hist, [byte_of(key, 0)], one16)
                    else:
                        m = byte_of(key, 0) == sel[0]
                        for q in range(1, p):
                            m = jnp.logical_and(m, byte_of(key, q) == sel[q])
                        plsc.addupdate_scatter(hist, [byte_of(key, p)], one16,
                                               mask=m)
                    return c

                lax.fori_loop(0, NV, hst, 0)

                ksp = spl(kth)

                def cross(vv, carry):
                    cnt, bstar, cabove, found = carry
                    v = 15 - vv
                    rv = jnp.flip(hist[pl.ds(v * 16, 16)])
                    c = plsc.cumsum(rv) + spl(cnt)
                    m = c >= ksp
                    lane = jnp.min(jnp.where(m, iota, spl(99)))
                    has = lane < 99
                    c_at = jnp.min(jnp.where(m, c, spl(1 << 30)))
                    rv_at = jnp.sum(jnp.where(spl(lane) == iota, rv, spl(0)))
                    take = jnp.logical_and(has, jnp.logical_not(found))
                    bin_cand = v * 16 + 15 - lane
                    bstar = jnp.where(take, bin_cand, bstar)
                    cabove = jnp.where(take, c_at - rv_at, cabove)
                    found = jnp.logical_or(found, has)
                    cnt = cnt + jnp.sum(rv)
                    return cnt, bstar, cabove, found

                _cnt_f, bstar, cabove, _fnd = lax.fori_loop(
                    0, 16, cross,
                    (jnp.int32(0), jnp.int32(0), jnp.int32(0),
                     jnp.zeros((), jnp.bool_)))
                sel.append(spl(bstar))
                kth = kth - cabove

            th = (((sel[0] ^ 0x80) << 24) | (sel[1] << 16)
                  | (sel[2] << 8) | sel[3])
            th_f = lax.bitcast_convert_type(th ^ ((th >> 31) & jnp.int32(0x7FFFFFFF)),
                                            jnp.float32)

            def gtp(v, off):
                key = keys[pl.ds(v * 16, 16)]
                m = key > th
                plsc.store_compressed(seli.at[ph, pl.ds(off, 16)],
                                      iota + spl(v * 16), mask=m)
                plsc.store_compressed(selv.at[ph, pl.ds(off, 16)],
                                      rowf[ph, pl.ds(v * 16, 16)], mask=m)
                return off + jnp.sum(m.astype(jnp.int32))

            r_cnt = lax.fori_loop(0, NV, gtp, jnp.int32(0))

            def eqp(v, off):
                key = keys[pl.ds(v * 16, 16)]
                m = key == th
                plsc.store_compressed(seli.at[ph, pl.ds(off, 16)],
                                      iota + spl(v * 16), mask=m)
                plsc.store_compressed(selv.at[ph, pl.ds(off, 16)], th_f, mask=m)
                return off + jnp.sum(m.astype(jnp.int32))

            lax.fori_loop(0, NV, eqp, r_cnt)

            pltpu.async_copy(table_hbm.at[seli.at[ph, pl.ds(0, K)]],
                             rows_v.at[ph], sem_g).wait()
            pltpu.async_copy(rows_v.at[ph],
                             states_hbm.at[pl.ds(orow * K, K)], sem_out)
            pltpu.async_copy(selv.at[ph, pl.ds(0, K)],
                             vals_hbm.at[pl.ds(orow * K, K)], sem_out)
            return 1 - ph

        lax.fori_loop(0, per_w, do_row, 0)
        for ph in range(2):
            pltpu.make_async_copy(
                rows_v.at[ph], states_hbm.at[pl.ds(0, K)], sem_out).wait()
            pltpu.make_async_copy(
                selv.at[ph, pl.ds(0, K)], vals_hbm.at[pl.ds(0, K)],
                sem_out).wait()

    return k(emis_flat, table)


def _scan_body(u_ref, v_ref, vals_ref, seq_ref, pblk_ref,
               out_ref, alpha_ref, last_ref, mm_ref):
    t = pl.program_id(0)
    T = pl.num_programs(0)

    @pl.when(t == 0)
    def _init():
        a0 = vals_ref[0]
        alpha_ref[:, :] = a0
        last_ref[:, :] = a0

    @pl.when(t > 0)
    def _step():
        for b in range(B):
            mm_ref[pl.ds(b * K, K), :] = jax.lax.dot_general(
                u_ref[0, b], v_ref[0, b], (((1,), (1,)), ((), ())),
                preferred_element_type=jnp.float32)
        pblk = pblk_ref[:, :]
        alpha = alpha_ref[:, :]
        amax = jnp.max(alpha, axis=1, keepdims=True)
        arel = alpha - amax
        arel_t = arel.T
        tiled = jnp.broadcast_to(
            arel_t.reshape(1, K, B), (B, K, B)).reshape(B * K, B)
        acol = jnp.sum(tiled * pblk, axis=1, keepdims=True)
        mm = mm_ref[:, :] + acol
        m3 = jnp.max(mm.reshape(B, K, K), axis=1)
        mcol = jax.lax.dot_general(
            pblk, m3, (((1,), (0,)), ((), ())),
            precision=jax.lax.Precision.HIGHEST,
            preferred_element_type=jnp.float32)
        p = jnp.exp(mm - mcol)
        s = jax.lax.dot_general(
            pblk, p, (((0,), (0,)), ((), ())),
            precision=jax.lax.Precision.HIGHEST,
            preferred_element_type=jnp.float32)
        alpha_new = vals_ref[0] + amax + m3 + jnp.log(s)
        alpha_ref[:, :] = alpha_new
        mask = seq_ref[:, :] - 1 == t
        last_ref[:, :] = jnp.where(mask, alpha_new, last_ref[:, :])

    @pl.when(t == T - 1)
    def _fin():
        la = last_ref[:, :]
        m = jnp.max(la, axis=1, keepdims=True)
        lse = m + jnp.log(jnp.sum(jnp.exp(la - m), axis=1, keepdims=True))
        for b in range(B):
            out_ref[b, 0] = lse[b, 0]


def _run_scan(states_tb, vals_tb, seq_lens, interpret=False):
    T, Bs, Ks, Es = states_tb.shape
    pblk = (jax.lax.broadcasted_iota(jnp.int32, (Bs * Ks, Bs), 0) // Ks
            == jax.lax.broadcasted_iota(jnp.int32, (Bs * Ks, Bs), 1)
            ).astype(jnp.float32)

    def im_u(t):
        return (jnp.maximum(t - 1, 0), 0, 0, 0)

    return pl.pallas_call(
        _scan_body,
        grid=(T,),
        in_specs=[
            pl.BlockSpec((1, Bs, Ks, Es), im_u),
            pl.BlockSpec((1, Bs, Ks, Es), lambda t: (t, 0, 0, 0)),
            pl.BlockSpec((1, Bs, Ks), lambda t: (t, 0, 0)),
            pl.BlockSpec((Bs, 1), lambda t: (0, 0)),
            pl.BlockSpec((Bs * Ks, Bs), lambda t: (0, 0)),
        ],
        out_specs=pl.BlockSpec(memory_space=pltpu.SMEM),
        out_shape=jax.ShapeDtypeStruct((Bs, 1), jnp.float32),
        scratch_shapes=[
            pltpu.VMEM((Bs, Ks), jnp.float32),
            pltpu.VMEM((Bs, Ks), jnp.float32),
            pltpu.VMEM((Bs * Ks, Ks), jnp.float32),
        ],
        interpret=interpret,
    )(states_tb, states_tb, vals_tb, seq_lens.reshape(Bs, 1), pblk)


@jax.jit
def kernel(state_matrix, emission_potentials, seq_lens, sum_size):
    Bs, T, Nn = emission_potentials.shape
    vals_flat, states_flat = _sc_topk_gather(
        emission_potentials.reshape(-1), state_matrix, T)
    vals_tb = vals_flat.reshape(T, Bs, K)
    states_tb = states_flat.reshape(T, Bs, K, E)
    out = _run_scan(states_tb, vals_tb, seq_lens)
    return out.reshape(Bs)


# speedup vs baseline: 2.6490x; 1.0057x over previous
import functools

import jax
import jax.numpy as jnp
from jax import lax
from jax.experimental import pallas as pl
from jax.experimental.pallas import tpu as pltpu
from jax.experimental.pallas import tpu_sc as plsc

K = 64
B = 32
N = 2048
E = 128

_NC = 2
_NS = 16
_NW = _NC * _NS


def _sc_topk_gather(emis_flat, table, T):
    R = emis_flat.shape[0] // N
    per_w = R // _NW
    NV = N // 16
    mesh = plsc.VectorSubcoreMesh(core_axis_name="c", subcore_axis_name="s")

    @functools.partial(
        pl.kernel, mesh=mesh,
        compiler_params=pltpu.CompilerParams(needs_layout_passes=False),
        out_type=[
            jax.ShapeDtypeStruct((R * K,), jnp.float32),
            jax.ShapeDtypeStruct((R * K, E), jnp.float32),
        ],
        scratch_types=[
            pltpu.VMEM((2, N), jnp.float32),
            pltpu.VMEM((N,), jnp.int32),
            pltpu.VMEM((256,), jnp.int32),
            pltpu.VMEM((2, N + 128), jnp.float32),
            pltpu.VMEM((2, N + 128), jnp.int32),
            pltpu.VMEM((2, K, E), jnp.float32),
            pltpu.SemaphoreType.DMA,
            pltpu.SemaphoreType.DMA,
            pltpu.SemaphoreType.DMA,
        ],
    )
    def k(emis_hbm, table_hbm, vals_hbm, states_hbm,
          rowf, keys, hist, selv, seli, rows_v, sem_in, sem_g, sem_out):
        wid = lax.axis_index("s") * _NC + lax.axis_index("c")
        iota = lax.iota(jnp.int32, 16)

        def spl(x, dt=jnp.int32):
            return lax.broadcast_in_dim(jnp.asarray(x, dt), (16,), ())

        zero16 = jnp.zeros((16,), jnp.int32)
        one16 = jnp.ones((16,), jnp.int32)
        row0 = wid * per_w
        last_row = row0 + per_w - 1

        pltpu.async_copy(emis_hbm.at[pl.ds(row0 * N, N)], rowf.at[0], sem_in)

        def do_row(r, ph):
            row = row0 + r
            b_idx = row // T
            t_idx = row - b_idx * T
            orow = t_idx * B + b_idx

            @pl.when(r >= 2)
            def _drain_out():
                pltpu.make_async_copy(
                    rows_v.at[ph], states_hbm.at[pl.ds(0, K)], sem_out).wait()
                pltpu.make_async_copy(
                    selv.at[ph, pl.ds(0, K)], vals_hbm.at[pl.ds(0, K)],
                    sem_out).wait()

            pltpu.make_async_copy(
                emis_hbm.at[pl.ds(0, N)], rowf.at[ph], sem_in).wait()

            @pl.when(row < last_row)
            def _prefetch():
                pltpu.async_copy(
                    emis_hbm.at[pl.ds((row + 1) * N, N)], rowf.at[1 - ph],
                    sem_in)

            def mk(v, c):
                x = rowf[ph, pl.ds(v * 16, 16)]
                xi = lax.bitcast_convert_type(x, jnp.int32)
                keys[pl.ds(v * 16, 16)] = xi ^ ((xi >> 31) & jnp.int32(0x7FFFFFFF))
                return c

            lax.fori_loop(0, NV, mk, 0, unroll=8)

            def byte_of(key, p):
                bb = (key >> (24 - 8 * p)) & 0xFF
                if p == 0:
                    bb = bb ^ 0x80
                return bb

            sel = []
            kth = jnp.int32(K)
            for p in range(4):
                def zh(v, c):
                    hist[pl.ds(v * 16, 16)] = zero16
                    return c

                lax.fori_loop(0, 16, zh, 0, unroll=4)

                def hst(v, c, p=p):
                    key = keys[pl.ds(v * 16, 16)]
                    if p == 0:
                        plsc.addupdate_scatter(hist, [byte_of(key, 0)], one16)
                    else:
                        m = byte_of(key, 0) == sel[0]
                        for q in range(1, p):
                            m = jnp.logical_and(m, byte_of(key, q) == sel[q])
                        plsc.addupdate_scatter(hist, [byte_of(key, p)], one16,
                                               mask=m)
                    return c

                lax.fori_loop(0, NV, hst, 0, unroll=8)

                ksp = spl(kth)

                def cross(vv, carry):
                    cnt, bstar, cabove, found = carry
                    v = 15 - vv
                    rv = jnp.flip(hist[pl.ds(v * 16, 16)])
                    c = plsc.cumsum(rv) + spl(cnt)
                    m = c >= ksp
                    lane = jnp.min(jnp.where(m, iota, spl(99)))
                    has = lane < 99
                    c_at = jnp.min(jnp.where(m, c, spl(1 << 30)))
                    rv_at = jnp.sum(jnp.where(spl(lane) == iota, rv, spl(0)))
                    take = jnp.logical_and(has, jnp.logical_not(found))
                    bin_cand = v * 16 + 15 - lane
                    bstar = jnp.where(take, bin_cand, bstar)
                    cabove = jnp.where(take, c_at - rv_at, cabove)
                    found = jnp.logical_or(found, has)
                    cnt = cnt + jnp.sum(rv)
                    return cnt, bstar, cabove, found

                _cnt_f, bstar, cabove, _fnd = lax.fori_loop(
                    0, 16, cross,
                    (jnp.int32(0), jnp.int32(0), jnp.int32(0),
                     jnp.zeros((), jnp.bool_)))
                sel.append(spl(bstar))
                kth = kth - cabove

            th = (((sel[0] ^ 0x80) << 24) | (sel[1] << 16)
                  | (sel[2] << 8) | sel[3])
            th_f = lax.bitcast_convert_type(th ^ ((th >> 31) & jnp.int32(0x7FFFFFFF)),
                                            jnp.float32)

            def gtp(v, off):
                key = keys[pl.ds(v * 16, 16)]
                m = key > th
                plsc.store_compressed(seli.at[ph, pl.ds(off, 16)],
                                      iota + spl(v * 16), mask=m)
                plsc.store_compressed(selv.at[ph, pl.ds(off, 16)],
                                      rowf[ph, pl.ds(v * 16, 16)], mask=m)
                return off + jnp.sum(m.astype(jnp.int32))

            r_cnt = lax.fori_loop(0, NV, gtp, jnp.int32(0), unroll=8)

            def eqp(v, off):
                key = keys[pl.ds(v * 16, 16)]
                m = key == th
                plsc.store_compressed(seli.at[ph, pl.ds(off, 16)],
                                      iota + spl(v * 16), mask=m)
                plsc.store_compressed(selv.at[ph, pl.ds(off, 16)], th_f, mask=m)
                return off + jnp.sum(m.astype(jnp.int32))

            lax.fori_loop(0, NV, eqp, r_cnt, unroll=8)

            pltpu.async_copy(table_hbm.at[seli.at[ph, pl.ds(0, K)]],
                             rows_v.at[ph], sem_g).wait()
            pltpu.async_copy(rows_v.at[ph],
                             states_hbm.at[pl.ds(orow * K, K)], sem_out)
            pltpu.async_copy(selv.at[ph, pl.ds(0, K)],
                             vals_hbm.at[pl.ds(orow * K, K)], sem_out)
            return 1 - ph

        lax.fori_loop(0, per_w, do_row, 0)
        for ph in range(2):
            pltpu.make_async_copy(
                rows_v.at[ph], states_hbm.at[pl.ds(0, K)], sem_out).wait()
            pltpu.make_async_copy(
                selv.at[ph, pl.ds(0, K)], vals_hbm.at[pl.ds(0, K)],
                sem_out).wait()

    return k(emis_flat, table)


def _scan_body(u_ref, v_ref, vals_ref, seq_ref, pblk_ref,
               out_ref, alpha_ref, last_ref, mm_ref):
    t = pl.program_id(0)
    T = pl.num_programs(0)

    @pl.when(t == 0)
    def _init():
        a0 = vals_ref[0]
        alpha_ref[:, :] = a0
        last_ref[:, :] = a0

    @pl.when(t > 0)
    def _step():
        for b in range(B):
            mm_ref[pl.ds(b * K, K), :] = jax.lax.dot_general(
                u_ref[0, b], v_ref[0, b], (((1,), (1,)), ((), ())),
                preferred_element_type=jnp.float32)
        pblk = pblk_ref[:, :]
        alpha = alpha_ref[:, :]
        amax = jnp.max(alpha, axis=1, keepdims=True)
        arel = alpha - amax
        arel_t = arel.T
        tiled = jnp.broadcast_to(
            arel_t.reshape(1, K, B), (B, K, B)).reshape(B * K, B)
        acol = jnp.sum(tiled * pblk, axis=1, keepdims=True)
        mm = mm_ref[:, :] + acol
        m3 = jnp.max(mm.reshape(B, K, K), axis=1)
        mcol = jax.lax.dot_general(
            pblk, m3, (((1,), (0,)), ((), ())),
            precision=jax.lax.Precision.HIGHEST,
            preferred_element_type=jnp.float32)
        p = jnp.exp(mm - mcol)
        s = jax.lax.dot_general(
            pblk, p, (((0,), (0,)), ((), ())),
            precision=jax.lax.Precision.HIGHEST,
            preferred_element_type=jnp.float32)
        alpha_new = vals_ref[0] + amax + m3 + jnp.log(s)
        alpha_ref[:, :] = alpha_new
        mask = seq_ref[:, :] - 1 == t
        last_ref[:, :] = jnp.where(mask, alpha_new, last_ref[:, :])

    @pl.when(t == T - 1)
    def _fin():
        la = last_ref[:, :]
        m = jnp.max(la, axis=1, keepdims=True)
        lse = m + jnp.log(jnp.sum(jnp.exp(la - m), axis=1, keepdims=True))
        for b in range(B):
            out_ref[b, 0] = lse[b, 0]


def _run_scan(states_tb, vals_tb, seq_lens, interpret=False):
    T, Bs, Ks, Es = states_tb.shape
    pblk = (jax.lax.broadcasted_iota(jnp.int32, (Bs * Ks, Bs), 0) // Ks
            == jax.lax.broadcasted_iota(jnp.int32, (Bs * Ks, Bs), 1)
            ).astype(jnp.float32)

    def im_u(t):
        return (jnp.maximum(t - 1, 0), 0, 0, 0)

    return pl.pallas_call(
        _scan_body,
        grid=(T,),
        in_specs=[
            pl.BlockSpec((1, Bs, Ks, Es), im_u),
            pl.BlockSpec((1, Bs, Ks, Es), lambda t: (t, 0, 0, 0)),
            pl.BlockSpec((1, Bs, Ks), lambda t: (t, 0, 0)),
            pl.BlockSpec((Bs, 1), lambda t: (0, 0)),
            pl.BlockSpec((Bs * Ks, Bs), lambda t: (0, 0)),
        ],
        out_specs=pl.BlockSpec(memory_space=pltpu.SMEM),
        out_shape=jax.ShapeDtypeStruct((Bs, 1), jnp.float32),
        scratch_shapes=[
            pltpu.VMEM((Bs, Ks), jnp.float32),
            pltpu.VMEM((Bs, Ks), jnp.float32),
            pltpu.VMEM((Bs * Ks, Ks), jnp.float32),
        ],
        interpret=interpret,
    )(states_tb, states_tb, vals_tb, seq_lens.reshape(Bs, 1), pblk)


@jax.jit
def kernel(state_matrix, emission_potentials, seq_lens, sum_size):
    Bs, T, Nn = emission_potentials.shape
    vals_flat, states_flat = _sc_topk_gather(
        emission_potentials.reshape(-1), state_matrix, T)
    vals_tb = vals_flat.reshape(T, Bs, K)
    states_tb = states_flat.reshape(T, Bs, K, E)
    out = _run_scan(states_tb, vals_tb, seq_lens)
    return out.reshape(Bs)
